# baseline (speedup 1.0000x reference)
import functools

import jax
import jax.numpy as jnp
from jax import lax
from jax.experimental import pallas as pl
from jax.experimental.pallas import tpu as pltpu
from jax.experimental.pallas import tpu_sc as plsc

BATCH = 4
SEQ = 8192
DIM = 1024

_info = plsc.get_sparse_core_info()
NC, NS = _info.num_cores, _info.num_subcores
NW = NC * NS
ROWS_PER_W = SEQ // NW
CHUNK = 64
N_CHUNKS = ROWS_PER_W // CHUNK

_mesh = plsc.VectorSubcoreMesh(core_axis_name="c", subcore_axis_name="s")


@functools.partial(
    pl.kernel,
    mesh=_mesh,
    out_type=jax.ShapeDtypeStruct((BATCH, SEQ, DIM), jnp.float32),
    scratch_types=[
        pltpu.VMEM((CHUNK, DIM), jnp.float32),
        pltpu.SemaphoreType.DMA,
    ],
)
def _broadcast_rows(table_hbm, out_hbm, buf, sem_d):
    wid = lax.axis_index("s") * NC + lax.axis_index("c")
    base = wid * ROWS_PER_W

    dh = pltpu.async_copy(
        table_hbm.at[pl.ds(base, ROWS_PER_W)],
        out_hbm.at[BATCH - 1, pl.ds(base, ROWS_PER_W)],
        sem_d,
    )

    for i in range(N_CHUNKS):
        r = base + i * CHUNK
        pltpu.sync_copy(table_hbm.at[pl.ds(r, CHUNK)], buf)
        for b in range(BATCH - 1):
            pltpu.sync_copy(buf, out_hbm.at[b, pl.ds(r, CHUNK)])

    dh.wait()


def kernel(x, pos_embedding):
    del x
    return _broadcast_rows(pos_embedding)


# speedup vs baseline: 14.3214x; 14.3214x over previous
import functools

import jax
import jax.numpy as jnp
from jax import lax
from jax.experimental import pallas as pl
from jax.experimental.pallas import tpu as pltpu
from jax.experimental.pallas import tpu_sc as plsc

BATCH = 4
SEQ = 8192
DIM = 1024

_info = plsc.get_sparse_core_info()
NC, NS = _info.num_cores, _info.num_subcores
NW = NC * NS
ROWS_PER_W = SEQ // NW
CHUNKS = (120, 120, 16)

_mesh = plsc.VectorSubcoreMesh(core_axis_name="c", subcore_axis_name="s")


@functools.partial(
    pl.kernel,
    mesh=_mesh,
    out_type=jax.ShapeDtypeStruct((BATCH, SEQ, DIM), jnp.float32),
    scratch_types=[
        pltpu.VMEM((CHUNKS[0], DIM), jnp.float32),
    ],
)
def _broadcast_rows(table_hbm, out_hbm, buf):
    wid = lax.axis_index("s") * NC + lax.axis_index("c")
    base = wid * ROWS_PER_W

    off = 0
    for sz in CHUNKS:
        r = base + off
        dst_buf = buf if sz == CHUNKS[0] else buf.at[pl.ds(0, sz)]
        pltpu.sync_copy(table_hbm.at[pl.ds(r, sz)], dst_buf)
        for b in range(BATCH):
            pltpu.sync_copy(dst_buf, out_hbm.at[b, pl.ds(r, sz)])
        off += sz


def kernel(x, pos_embedding):
    del x
    return _broadcast_rows(pos_embedding)
